# baseline (speedup 1.0000x reference)
import functools

import jax
import jax.numpy as jnp
from jax import lax
from jax.experimental import pallas as pl
from jax.experimental.pallas import tpu as pltpu
from jax.experimental.pallas import tpu_sc as plsc

_NUM_TYPES = 8
_IN_F = 1024
_OUT_F = 1024
_N = 2048

_BLK = 128
_NB = _N // _BLK + _NUM_TYPES
_TOTAL = _NB * _BLK

_NC = 2
_NS = 16
_NW = _NC * _NS


@functools.lru_cache(maxsize=None)
def _make_sc_dispatch(num_rows, num_slots, feat):
    n_per_w = num_rows // _NW
    assert num_rows % _NW == 0 and n_per_w % 8 == 0
    mesh = plsc.VectorSubcoreMesh(
        core_axis_name="c", subcore_axis_name="s",
        num_cores=_NC, num_subcores=_NS)

    @functools.partial(
        pl.kernel, mesh=mesh,
        out_type=jax.ShapeDtypeStruct((num_slots, feat), jnp.float32),
        scratch_types=[
            pltpu.VMEM((n_per_w,), jnp.int32),
            pltpu.VMEM((n_per_w, feat), jnp.float32),
            pltpu.SemaphoreType.DMA,
        ],
    )
    def dispatch_k(x_hbm, slot_hbm, out_hbm, idx_v, rows_v, sem):
        wid = lax.axis_index("s") * _NC + lax.axis_index("c")
        base = wid * n_per_w
        pltpu.sync_copy(slot_hbm.at[pl.ds(base, n_per_w)], idx_v)
        pltpu.sync_copy(x_hbm.at[pl.ds(base, n_per_w)], rows_v)
        pltpu.async_copy(rows_v, out_hbm.at[idx_v], sem).wait()

    return dispatch_k


@functools.lru_cache(maxsize=None)
def _make_sc_gather(num_idx, feat):
    b_per_w = num_idx // _NW
    assert num_idx % _NW == 0 and b_per_w % 8 == 0
    mesh = plsc.VectorSubcoreMesh(
        core_axis_name="c", subcore_axis_name="s",
        num_cores=_NC, num_subcores=_NS)

    @functools.partial(
        pl.kernel, mesh=mesh,
        out_type=jax.ShapeDtypeStruct((num_idx, feat), jnp.float32),
        scratch_types=[
            pltpu.VMEM((b_per_w,), jnp.int32),
            pltpu.VMEM((b_per_w, feat), jnp.float32),
            pltpu.SemaphoreType.DMA,
        ],
    )
    def gather_k(table_hbm, idx_hbm, out_hbm, idx_v, rows_v, sem):
        wid = lax.axis_index("s") * _NC + lax.axis_index("c")
        base = wid * b_per_w
        pltpu.sync_copy(idx_hbm.at[pl.ds(base, b_per_w)], idx_v)
        pltpu.async_copy(table_hbm.at[idx_v], rows_v, sem).wait()
        pltpu.sync_copy(rows_v, out_hbm.at[pl.ds(base, b_per_w)])

    return gather_k


def _mm_body(s_ref, x_ref, w_ref, b_ref, o_ref):
    i = pl.program_id(0)

    @pl.when(i < s_ref[3 * _NB])
    def _():
        t = s_ref[i]
        acc = lax.dot_general(
            x_ref[...], w_ref[t],
            (((1,), (1,)), ((), ())),
            preferred_element_type=jnp.float32,
        )
        o_ref[...] = acc + b_ref[t]


_grid_spec = pltpu.PrefetchScalarGridSpec(
    num_scalar_prefetch=1,
    grid=(_NB,),
    in_specs=[
        pl.BlockSpec((_BLK, _IN_F), lambda i, s: (s[_NB + i], 0)),
        pl.BlockSpec((_NUM_TYPES, _OUT_F, _IN_F), lambda i, s: (0, 0, 0)),
        pl.BlockSpec((_NUM_TYPES, 1, _OUT_F), lambda i, s: (0, 0, 0)),
    ],
    out_specs=pl.BlockSpec((_BLK, _OUT_F), lambda i, s: (s[2 * _NB + i], 0)),
)


def kernel(x, types, W, b):
    types = types.astype(jnp.int32)
    oh = (types[None, :] ==
          jnp.arange(_NUM_TYPES, dtype=jnp.int32)[:, None]).astype(jnp.int32)
    csum = jnp.cumsum(oh, axis=1)
    counts = csum[:, -1]
    padded = ((counts + _BLK - 1) // _BLK) * _BLK
    pstart = jnp.concatenate(
        [jnp.zeros((1,), jnp.int32), jnp.cumsum(padded)])
    slot = jnp.sum((pstart[:_NUM_TYPES, None] - 1 + csum) * oh, axis=0)
    nact = pstart[_NUM_TYPES] // _BLK
    blk_start = jnp.arange(_NB, dtype=jnp.int32) * _BLK
    bt = jnp.sum((pstart[1:_NUM_TYPES, None] <= blk_start[None, :])
                 .astype(jnp.int32), axis=0)
    bidx = jnp.arange(_NB, dtype=jnp.int32)
    last = jnp.sum(bt * (bidx == jnp.maximum(nact - 1, 0)))
    active = blk_start < pstart[_NUM_TYPES]
    bt = jnp.where(active, bt, last)
    xi = jnp.where(active, bidx, jnp.maximum(nact - 1, 0))
    oi = jnp.where(active, bidx, _NB - 1)
    scalars = jnp.concatenate([bt, xi, oi, nact[None]])

    x_pad = _make_sc_dispatch(_N, _TOTAL, _IN_F)(x, slot)
    out_pad = pl.pallas_call(
        _mm_body,
        grid_spec=_grid_spec,
        out_shape=jax.ShapeDtypeStruct((_TOTAL, _OUT_F), jnp.float32),
    )(scalars, x_pad, W, b.reshape(_NUM_TYPES, 1, _OUT_F))
    return _make_sc_gather(_N, _OUT_F)(out_pad, slot)


# speedup vs baseline: 1.0860x; 1.0860x over previous
import functools

import jax
import jax.numpy as jnp
from jax import lax
from jax.experimental import pallas as pl
from jax.experimental.pallas import tpu as pltpu
from jax.experimental.pallas import tpu_sc as plsc

_NUM_TYPES = 8
_IN_F = 1024
_OUT_F = 1024
_N = 2048

_BLK = 256
_NB = _N // _BLK + _NUM_TYPES
_TOTAL = _NB * _BLK

_NC = 2
_NS = 16
_NW = _NC * _NS


@functools.lru_cache(maxsize=None)
def _make_sc_dispatch(num_rows, num_slots, feat):
    n_per_w = num_rows // _NW
    assert num_rows % _NW == 0 and n_per_w % 8 == 0
    mesh = plsc.VectorSubcoreMesh(
        core_axis_name="c", subcore_axis_name="s",
        num_cores=_NC, num_subcores=_NS)

    @functools.partial(
        pl.kernel, mesh=mesh,
        out_type=jax.ShapeDtypeStruct((num_slots, feat), jnp.float32),
        scratch_types=[
            pltpu.VMEM((n_per_w,), jnp.int32),
            pltpu.VMEM((n_per_w, feat), jnp.float32),
            pltpu.SemaphoreType.DMA,
        ],
    )
    def dispatch_k(x_hbm, slot_hbm, out_hbm, idx_v, rows_v, sem):
        wid = lax.axis_index("s") * _NC + lax.axis_index("c")
        base = wid * n_per_w
        pltpu.sync_copy(slot_hbm.at[pl.ds(base, n_per_w)], idx_v)
        pltpu.sync_copy(x_hbm.at[pl.ds(base, n_per_w)], rows_v)
        pltpu.async_copy(rows_v, out_hbm.at[idx_v], sem).wait()

    return dispatch_k


@functools.lru_cache(maxsize=None)
def _make_sc_gather(num_idx, feat):
    b_per_w = num_idx // _NW
    assert num_idx % _NW == 0 and b_per_w % 8 == 0
    mesh = plsc.VectorSubcoreMesh(
        core_axis_name="c", subcore_axis_name="s",
        num_cores=_NC, num_subcores=_NS)

    @functools.partial(
        pl.kernel, mesh=mesh,
        out_type=jax.ShapeDtypeStruct((num_idx, feat), jnp.float32),
        scratch_types=[
            pltpu.VMEM((b_per_w,), jnp.int32),
            pltpu.VMEM((b_per_w, feat), jnp.float32),
            pltpu.SemaphoreType.DMA,
        ],
    )
    def gather_k(table_hbm, idx_hbm, out_hbm, idx_v, rows_v, sem):
        wid = lax.axis_index("s") * _NC + lax.axis_index("c")
        base = wid * b_per_w
        pltpu.sync_copy(idx_hbm.at[pl.ds(base, b_per_w)], idx_v)
        pltpu.async_copy(table_hbm.at[idx_v], rows_v, sem).wait()
        pltpu.sync_copy(rows_v, out_hbm.at[pl.ds(base, b_per_w)])

    return gather_k


def _mm_body(s_ref, x_ref, w_ref, b_ref, o_ref):
    i = pl.program_id(0)

    @pl.when(i < s_ref[3 * _NB])
    def _():
        t = s_ref[i]
        acc = lax.dot_general(
            x_ref[...], w_ref[t],
            (((1,), (1,)), ((), ())),
            preferred_element_type=jnp.float32,
        )
        o_ref[...] = acc + b_ref[t]


_grid_spec = pltpu.PrefetchScalarGridSpec(
    num_scalar_prefetch=1,
    grid=(_NB,),
    in_specs=[
        pl.BlockSpec((_BLK, _IN_F), lambda i, s: (s[_NB + i], 0)),
        pl.BlockSpec((_NUM_TYPES, _OUT_F, _IN_F), lambda i, s: (0, 0, 0)),
        pl.BlockSpec((_NUM_TYPES, 1, _OUT_F), lambda i, s: (0, 0, 0)),
    ],
    out_specs=pl.BlockSpec((_BLK, _OUT_F), lambda i, s: (s[2 * _NB + i], 0)),
)


def kernel(x, types, W, b):
    types = types.astype(jnp.int32)
    oh = (types[None, :] ==
          jnp.arange(_NUM_TYPES, dtype=jnp.int32)[:, None]).astype(jnp.int32)
    csum = jnp.cumsum(oh, axis=1)
    counts = csum[:, -1]
    padded = ((counts + _BLK - 1) // _BLK) * _BLK
    pstart = jnp.concatenate(
        [jnp.zeros((1,), jnp.int32), jnp.cumsum(padded)])
    slot = jnp.sum((pstart[:_NUM_TYPES, None] - 1 + csum) * oh, axis=0)
    nact = pstart[_NUM_TYPES] // _BLK
    blk_start = jnp.arange(_NB, dtype=jnp.int32) * _BLK
    bt = jnp.sum((pstart[1:_NUM_TYPES, None] <= blk_start[None, :])
                 .astype(jnp.int32), axis=0)
    bidx = jnp.arange(_NB, dtype=jnp.int32)
    last = jnp.sum(bt * (bidx == jnp.maximum(nact - 1, 0)))
    active = blk_start < pstart[_NUM_TYPES]
    bt = jnp.where(active, bt, last)
    xi = jnp.where(active, bidx, jnp.maximum(nact - 1, 0))
    oi = jnp.where(active, bidx, _NB - 1)
    scalars = jnp.concatenate([bt, xi, oi, nact[None]])

    x_pad = _make_sc_dispatch(_N, _TOTAL, _IN_F)(x, slot)
    out_pad = pl.pallas_call(
        _mm_body,
        grid_spec=_grid_spec,
        out_shape=jax.ShapeDtypeStruct((_TOTAL, _OUT_F), jnp.float32),
    )(scalars, x_pad, W, b.reshape(_NUM_TYPES, 1, _OUT_F))
    return _make_sc_gather(_N, _OUT_F)(out_pad, slot)


# speedup vs baseline: 1.0926x; 1.0061x over previous
import functools

import jax
import jax.numpy as jnp
from jax import lax
from jax.experimental import pallas as pl
from jax.experimental.pallas import tpu as pltpu
from jax.experimental.pallas import tpu_sc as plsc

_NUM_TYPES = 8
_IN_F = 1024
_OUT_F = 1024
_N = 2048

_BLK = 256
_NB = _N // _BLK + _NUM_TYPES
_TOTAL = _NB * _BLK

_NC = 2
_NS = 16
_NW = _NC * _NS


@functools.lru_cache(maxsize=None)
def _make_sc_dispatch(num_rows, num_slots, feat):
    n_per_w = num_rows // _NW
    assert num_rows % _NW == 0 and n_per_w % 8 == 0
    mesh = plsc.VectorSubcoreMesh(
        core_axis_name="c", subcore_axis_name="s",
        num_cores=_NC, num_subcores=_NS)

    @functools.partial(
        pl.kernel, mesh=mesh,
        out_type=jax.ShapeDtypeStruct((num_slots, feat), jnp.float32),
        scratch_types=[
            pltpu.VMEM((n_per_w,), jnp.int32),
            pltpu.VMEM((n_per_w, feat), jnp.float32),
            pltpu.SemaphoreType.DMA,
        ],
    )
    def dispatch_k(x_hbm, slot_hbm, out_hbm, idx_v, rows_v, sem):
        wid = lax.axis_index("s") * _NC + lax.axis_index("c")
        base = wid * n_per_w
        pltpu.sync_copy(slot_hbm.at[pl.ds(base, n_per_w)], idx_v)
        pltpu.sync_copy(x_hbm.at[pl.ds(base, n_per_w)], rows_v)
        pltpu.async_copy(rows_v, out_hbm.at[idx_v], sem).wait()

    return dispatch_k


@functools.lru_cache(maxsize=None)
def _make_sc_gather(num_idx, feat):
    b_per_w = num_idx // _NW
    assert num_idx % _NW == 0 and b_per_w % 8 == 0
    mesh = plsc.VectorSubcoreMesh(
        core_axis_name="c", subcore_axis_name="s",
        num_cores=_NC, num_subcores=_NS)

    @functools.partial(
        pl.kernel, mesh=mesh,
        out_type=jax.ShapeDtypeStruct((num_idx, feat), jnp.float32),
        scratch_types=[
            pltpu.VMEM((b_per_w,), jnp.int32),
            pltpu.VMEM((b_per_w, feat), jnp.float32),
            pltpu.SemaphoreType.DMA,
        ],
    )
    def gather_k(table_hbm, idx_hbm, out_hbm, idx_v, rows_v, sem):
        wid = lax.axis_index("s") * _NC + lax.axis_index("c")
        base = wid * b_per_w
        pltpu.sync_copy(idx_hbm.at[pl.ds(base, b_per_w)], idx_v)
        pltpu.async_copy(table_hbm.at[idx_v], rows_v, sem).wait()
        pltpu.sync_copy(rows_v, out_hbm.at[pl.ds(base, b_per_w)])

    return gather_k


def _mm_body(s_ref, x_ref, w_ref, b_ref, o_ref):
    i = pl.program_id(0)

    @pl.when(i < s_ref[3 * _NB])
    def _():
        acc = lax.dot_general(
            x_ref[...], w_ref[0],
            (((1,), (1,)), ((), ())),
            preferred_element_type=jnp.float32,
        )
        o_ref[...] = acc + b_ref[0]


_grid_spec = pltpu.PrefetchScalarGridSpec(
    num_scalar_prefetch=1,
    grid=(_NB,),
    in_specs=[
        pl.BlockSpec((_BLK, _IN_F), lambda i, s: (s[_NB + i], 0)),
        pl.BlockSpec((1, _OUT_F, _IN_F), lambda i, s: (s[i], 0, 0)),
        pl.BlockSpec((1, 1, _OUT_F), lambda i, s: (s[i], 0, 0)),
    ],
    out_specs=pl.BlockSpec((_BLK, _OUT_F), lambda i, s: (s[2 * _NB + i], 0)),
)


def kernel(x, types, W, b):
    types = types.astype(jnp.int32)
    oh = (types[None, :] ==
          jnp.arange(_NUM_TYPES, dtype=jnp.int32)[:, None]).astype(jnp.int32)
    csum = jnp.cumsum(oh, axis=1)
    counts = csum[:, -1]
    padded = ((counts + _BLK - 1) // _BLK) * _BLK
    pstart = jnp.concatenate(
        [jnp.zeros((1,), jnp.int32), jnp.cumsum(padded)])
    slot = jnp.sum((pstart[:_NUM_TYPES, None] - 1 + csum) * oh, axis=0)
    nact = pstart[_NUM_TYPES] // _BLK
    blk_start = jnp.arange(_NB, dtype=jnp.int32) * _BLK
    bt = jnp.sum((pstart[1:_NUM_TYPES, None] <= blk_start[None, :])
                 .astype(jnp.int32), axis=0)
    bidx = jnp.arange(_NB, dtype=jnp.int32)
    last = jnp.sum(bt * (bidx == jnp.maximum(nact - 1, 0)))
    active = blk_start < pstart[_NUM_TYPES]
    bt = jnp.where(active, bt, last)
    xi = jnp.where(active, bidx, jnp.maximum(nact - 1, 0))
    oi = jnp.where(active, bidx, _NB - 1)
    scalars = jnp.concatenate([bt, xi, oi, nact[None]])

    x_pad = _make_sc_dispatch(_N, _TOTAL, _IN_F)(x, slot)
    out_pad = pl.pallas_call(
        _mm_body,
        grid_spec=_grid_spec,
        out_shape=jax.ShapeDtypeStruct((_TOTAL, _OUT_F), jnp.float32),
    )(scalars, x_pad, W, b.reshape(_NUM_TYPES, 1, _OUT_F))
    return _make_sc_gather(_N, _OUT_F)(out_pad, slot)
